# baseline (speedup 1.0000x reference)
import functools

import jax
import jax.numpy as jnp
from jax import lax
from jax.experimental import pallas as pl
from jax.experimental.pallas import tpu as pltpu
from jax.experimental.pallas import tpu_sc as plsc

NG = 64
DELTA = 20.0 / (NG - 1)
COEFF = -0.5 / (DELTA * DELTA)
LANES = 16
NW = 32
SUB = 100
CHUNK = 400
NBUF = 2


@functools.lru_cache(maxsize=None)
def _build(E):
    per_w = E // NW
    n_chunks = per_w // CHUNK
    n_sub = CHUNK // SUB
    n_outer = n_chunks // NBUF
    mesh = plsc.VectorSubcoreMesh(
        core_axis_name="c", subcore_axis_name="s", num_cores=2, num_subcores=16
    )

    @functools.partial(
        pl.kernel,
        out_type=jax.ShapeDtypeStruct((E, 2 * NG), jnp.float32),
        mesh=mesh,
        compiler_params=pltpu.CompilerParams(
            use_tc_tiling_on_sc=False, needs_layout_passes=False
        ),
        scratch_types=[
            pltpu.VMEM((NBUF, CHUNK), jnp.float32),
            pltpu.VMEM((NBUF, n_sub, SUB), jnp.int32),
            pltpu.VMEM((NBUF, CHUNK, NG), jnp.float32),
            pltpu.VMEM((NBUF, CHUNK, NG), jnp.float32),
            pltpu.SemaphoreType.DMA,
            pltpu.SemaphoreType.DMA,
        ],
    )
    def sc_kernel(len_hbm, idx_hbm, table_hbm, out_hbm,
                  len_v, idx_v, emb_v, rbf_v, gat_sem, out_sems):
        wid = lax.axis_index("s") * 2 + lax.axis_index("c")
        lane = lax.iota(jnp.int32, LANES)

        def process(c, b, wait_prev):
            base = wid * per_w + c * CHUNK
            rbf_b, emb_b = rbf_v.at[b], emb_v.at[b]
            out_rbf = out_hbm.at[pl.ds(base, CHUNK), pl.ds(0, NG)]
            out_emb = out_hbm.at[pl.ds(base, CHUNK), pl.ds(NG, NG)]
            @pl.when(wait_prev)
            def _():
                pltpu.make_async_copy(rbf_b, out_rbf, out_sems).wait()
                pltpu.make_async_copy(emb_b, out_emb, out_sems).wait()

            pltpu.sync_copy(len_hbm.at[pl.ds(base, CHUNK)], len_v.at[b])
            pltpu.sync_copy(idx_hbm.at[pl.ds(base // SUB, n_sub)], idx_v.at[b])
            gathers = [
                pltpu.async_copy(
                    table_hbm.at[idx_v.at[b, j]],
                    emb_b.at[pl.ds(j * SUB, SUB)],
                    gat_sem,
                )
                for j in range(n_sub)
            ]

            def e_body(e, carry2):
                d16 = len_v[b, pl.ds(e * LANES, LANES)]
                row = lane + e * LANES
                for g in range(NG):
                    t = d16 - (g * DELTA)
                    v = jnp.exp(COEFF * (t * t))
                    col = jnp.full((LANES,), g, jnp.int32)
                    plsc.store_scatter(rbf_b, [row, col], v)
                return carry2

            lax.fori_loop(0, CHUNK // LANES, e_body, 0, unroll=False)
            for g_ in gathers:
                g_.wait()
            pltpu.async_copy(rbf_b, out_rbf, out_sems)
            pltpu.async_copy(emb_b, out_emb, out_sems)

        def outer(i, carry):
            for b in range(NBUF):
                process(i * NBUF + b, b, i > 0)
            return carry

        lax.fori_loop(0, n_outer, outer, 0, unroll=False)

        for b in range(NBUF):
            c = (n_outer - 1) * NBUF + b
            base = wid * per_w + c * CHUNK
            pltpu.make_async_copy(
                rbf_v.at[b], out_hbm.at[pl.ds(base, CHUNK), pl.ds(0, NG)],
                out_sems).wait()
            pltpu.make_async_copy(
                emb_v.at[b], out_hbm.at[pl.ds(base, CHUNK), pl.ds(NG, NG)],
                out_sems).wait()

    return sc_kernel


def kernel(edge_length, edge_type, bond_emb_weight):
    E = edge_length.shape[0]
    lengths = edge_length.reshape(E)
    idx = edge_type.astype(jnp.int32).reshape(E // SUB, SUB)
    fn = _build(E)
    return fn(lengths, idx, bond_emb_weight)
